# baseline (speedup 1.0000x reference)
import jax
import jax.numpy as jnp
from jax import lax
from jax.experimental import pallas as pl
from jax.experimental.pallas import tpu as pltpu
from jax.experimental.pallas import tpu_sc as plsc

N_C = 4096
M_F = 16384
D_IN = 512
D_SKIP = 256
D_OUT = 512
BM = 256
N_XW = N_C // BM

SC_LANES = 16
SC_NW = 32
SC_P = M_F // SC_NW
SC_CP = 16
SC_R = 4 * SC_CP


NCB = 512
N_J = N_C // NCB


def _insert(x, xi, v1, v2, v3, i1, i2, i3):
    c1 = x < v1
    c2 = x < v2
    c3 = x < v3
    nv1 = jnp.where(c1, x, v1)
    ni1 = jnp.where(c1, xi, i1)
    nv2 = jnp.where(c1, v1, jnp.where(c2, x, v2))
    ni2 = jnp.where(c1, i1, jnp.where(c2, xi, i2))
    nv3 = jnp.where(c2, v2, jnp.where(c3, x, v3))
    ni3 = jnp.where(c2, i2, jnp.where(c3, xi, i3))
    return nv1, nv2, nv3, ni1, ni2, ni3


def _tc_body(lo_ref, hi_ref, posT_ref, batchf_ref, x_ref, ps_ref, bsf_ref,
             xs_ref, w1t_ref, w2t_ref, b_ref, t_ref, idx_ref, wn_ref,
             v1_s, v2_s, v3_s, i1_s, i2_s, i3_s):
    i = pl.program_id(0)
    j = pl.program_id(1)
    lo_i = lo_ref[i]
    hi_i = hi_ref[i]
    active = jnp.logical_and(j >= lo_i, j < hi_i)

    @pl.when(jnp.logical_and(i < N_XW, j == 0))
    def _xw():
        t_ref[...] = jnp.dot(x_ref[...].astype(jnp.bfloat16), w1t_ref[...],
                             preferred_element_type=jnp.float32)

    @pl.when(jnp.logical_and(i >= N_XW, active))
    def _knn():
        @pl.when(j == lo_i)
        def _init():
            inf2 = jnp.full((BM, 1), jnp.inf, jnp.float32)
            v1_s[...] = inf2
            v2_s[...] = inf2
            v3_s[...] = inf2
            i1_s[...] = inf2
            i2_s[...] = inf2
            i3_s[...] = inf2
            t_ref[...] = (jnp.dot(xs_ref[...].astype(jnp.bfloat16),
                                  w2t_ref[...],
                                  preferred_element_type=jnp.float32)
                          + b_ref[...])

        q0 = ps_ref[:, 0:1]
        q1 = ps_ref[:, 1:2]
        q2 = ps_ref[:, 2:3]
        p0 = posT_ref[0:1, :]
        p1 = posT_ref[1:2, :]
        p2 = posT_ref[2:3, :]
        d2 = (q0 - p0) ** 2 + (q1 - p1) ** 2 + (q2 - p2) ** 2
        same = bsf_ref[...] == batchf_ref[...]
        masked = jnp.where(same, d2, jnp.inf)

        iota = (jax.lax.broadcasted_iota(jnp.int32, (1, NCB), 1)
                .astype(jnp.float32) + (j * NCB).astype(jnp.float32))
        v1 = v1_s[...]
        v2 = v2_s[...]
        v3 = v3_s[...]
        i1 = i1_s[...]
        i2 = i2_s[...]
        i3 = i3_s[...]
        for k in range(3):
            mval = jnp.min(masked, axis=1, keepdims=True)
            tie = jnp.where(masked == mval, iota, jnp.inf)
            midx = jnp.min(tie, axis=1, keepdims=True)
            v1, v2, v3, i1, i2, i3 = _insert(mval, midx,
                                             v1, v2, v3, i1, i2, i3)
            if k < 2:
                hit = iota == midx
                masked = jnp.where(hit, jnp.inf, masked)
        v1_s[...] = v1
        v2_s[...] = v2
        v3_s[...] = v3
        i1_s[...] = i1
        i2_s[...] = i2
        i3_s[...] = i3

        @pl.when(j == hi_i - 1)
        def _finalize():
            w1 = 1.0 / (jnp.sqrt(v1) + 1e-8)
            w2 = 1.0 / (jnp.sqrt(v2) + 1e-8)
            w3 = 1.0 / (jnp.sqrt(v3) + 1e-8)
            inv = 1.0 / (w1 + w2 + w3 + 1e-8)
            row0 = N_C + (i - N_XW) * BM
            self_rows = (row0
                         + jax.lax.broadcasted_iota(jnp.int32, (BM, 1), 0)
                         ).astype(jnp.float32)
            safe = lambda ix: jnp.where(jnp.isfinite(ix), ix, 0.0)
            idx_ref[...] = jnp.concatenate(
                [safe(i1), safe(i2), safe(i3), self_rows],
                axis=1).astype(jnp.int32)
            wn_ref[...] = jnp.concatenate(
                [w1 * inv, w2 * inv, w3 * inv,
                 jnp.ones((BM, 1), jnp.float32)], axis=1)


def _sc_body(t_hbm, idx_hbm, wrep_hbm, out_hbm,
             idx0, idx1, rows0, rows1, wrep0, wrep1, acc0, acc1,
             semg0, semg1, semo0, semo1, semi0, semi1, semw0, semw1):
    wid = lax.axis_index("s") * 2 + lax.axis_index("c")
    nchunk = SC_P // SC_CP
    half = nchunk // 2

    def aux_slice(g):
        return pl.ds(4 * (wid * SC_P + g * SC_CP), SC_R)

    def out_slice(g):
        return pl.ds(wid * SC_P + g * SC_CP, SC_CP)

    def compute(rows_v, wrep_v, acc_v):
        def point(p, c):
            w0 = wrep_v[4 * p, :]
            w1 = wrep_v[4 * p + 1, :]
            w2 = wrep_v[4 * p + 2, :]
            w3 = wrep_v[4 * p + 3, :]
            for v in range(D_OUT // SC_LANES):
                sl = pl.ds(SC_LANES * v, SC_LANES)
                acc_v[p, sl] = (w0 * rows_v[4 * p, sl]
                                + w1 * rows_v[4 * p + 1, sl]
                                + w2 * rows_v[4 * p + 2, sl]
                                + w3 * rows_v[4 * p + 3, sl])
            return c

        lax.fori_loop(0, SC_CP, point, 0)

    pltpu.sync_copy(idx_hbm.at[aux_slice(0)], idx0)
    pltpu.sync_copy(wrep_hbm.at[aux_slice(0)], wrep0)
    pltpu.async_copy(t_hbm.at[idx0], rows0, semg0)
    pltpu.sync_copy(idx_hbm.at[aux_slice(1)], idx1)
    pltpu.sync_copy(wrep_hbm.at[aux_slice(1)], wrep1)
    pltpu.async_copy(t_hbm.at[idx1], rows1, semg1)

    def phase(h, g, idx_v, rows_v, wrep_v, acc_v, semg, semo, semi, semw):
        @pl.when(h > 0)
        def _():
            pltpu.make_async_copy(acc_v, out_hbm.at[out_slice(g)], semo).wait()

        pltpu.make_async_copy(t_hbm.at[idx_v], rows_v, semg).wait()

        @pl.when(h < half - 1)
        def _():
            pltpu.async_copy(idx_hbm.at[aux_slice(g + 2)], idx_v, semi)

        @pl.when(h > 0)
        def _():
            pltpu.make_async_copy(wrep_hbm.at[aux_slice(g)], wrep_v, semw).wait()

        compute(rows_v, wrep_v, acc_v)
        pltpu.async_copy(acc_v, out_hbm.at[out_slice(g)], semo)

        @pl.when(h < half - 1)
        def _():
            pltpu.make_async_copy(idx_hbm.at[aux_slice(g + 2)], idx_v, semi).wait()
            pltpu.async_copy(t_hbm.at[idx_v], rows_v, semg)
            pltpu.async_copy(wrep_hbm.at[aux_slice(g + 2)], wrep_v, semw)

    def body(h, carry):
        phase(h, 2 * h, idx0, rows0, wrep0, acc0, semg0, semo0, semi0, semw0)
        phase(h, 2 * h + 1, idx1, rows1, wrep1, acc1, semg1, semo1, semi1, semw1)
        return carry

    lax.fori_loop(0, half, body, 0)
    pltpu.make_async_copy(acc0, out_hbm.at[out_slice(nchunk - 2)], semo0).wait()
    pltpu.make_async_copy(acc1, out_hbm.at[out_slice(nchunk - 1)], semo1).wait()


def _tc_stage(x, pos, batch, x_skip, pos_skip, batch_skip, W, b,
              interpret=False):
    posT = pos.T
    batchf = batch.astype(jnp.float32).reshape(1, N_C)
    bsf = batch_skip.astype(jnp.float32).reshape(M_F, 1)
    WT = W.T.astype(jnp.bfloat16)
    w1t = WT[:D_IN]
    w2t = WT[D_IN:]
    b2 = b.reshape(1, D_OUT)

    nfb = M_F // BM
    bvals = jnp.arange(8, dtype=batch.dtype)
    cs = jnp.searchsorted(batch, bvals, side="left").astype(jnp.int32)
    ce = jnp.searchsorted(batch, bvals, side="right").astype(jnp.int32)
    bs2 = batch_skip.reshape(nfb, BM)
    lo_f = (cs[bs2[:, 0]] // NCB).astype(jnp.int32)
    hi_f = ((ce[bs2[:, -1]] + NCB - 1) // NCB).astype(jnp.int32)
    lo_f = jnp.minimum(lo_f, N_J - 1)
    hi_f = jnp.minimum(jnp.maximum(hi_f, lo_f + 1), N_J)
    lo = jnp.concatenate([jnp.zeros((N_XW,), jnp.int32), lo_f])
    hi = jnp.concatenate([jnp.ones((N_XW,), jnp.int32), hi_f])

    fine = lambda i, j, lo_r, hi_r: (jnp.maximum(i - N_XW, 0), 0)
    cst = lambda i, j, lo_r, hi_r: (0, 0)
    t, idx, wn = pl.pallas_call(
        _tc_body,
        grid_spec=pltpu.PrefetchScalarGridSpec(
            num_scalar_prefetch=2,
            grid=(N_XW + nfb, N_J),
            in_specs=[
                pl.BlockSpec((3, NCB), lambda i, j, lo_r, hi_r: (0, j)),
                pl.BlockSpec((1, NCB), lambda i, j, lo_r, hi_r: (0, j)),
                pl.BlockSpec((BM, D_IN),
                             lambda i, j, lo_r, hi_r:
                             (jnp.minimum(i, N_XW - 1), 0)),
                pl.BlockSpec((BM, 3), fine),
                pl.BlockSpec((BM, 1), fine),
                pl.BlockSpec((BM, D_SKIP), fine),
                pl.BlockSpec((D_IN, D_OUT), cst),
                pl.BlockSpec((D_SKIP, D_OUT), cst),
                pl.BlockSpec((1, D_OUT), cst),
            ],
            out_specs=[
                pl.BlockSpec((BM, D_OUT), lambda i, j, lo_r, hi_r: (i, 0)),
                pl.BlockSpec((BM, 4), fine),
                pl.BlockSpec((BM, 4), fine),
            ],
            scratch_shapes=[pltpu.VMEM((BM, 1), jnp.float32)] * 6,
        ),
        out_shape=[
            jax.ShapeDtypeStruct((N_C + M_F, D_OUT), jnp.float32),
            jax.ShapeDtypeStruct((M_F, 4), jnp.int32),
            jax.ShapeDtypeStruct((M_F, 4), jnp.float32),
        ],
        interpret=interpret,
    )(lo, hi, posT, batchf, x, pos_skip, bsf, x_skip, w1t, w2t, b2)
    return t, idx, wn


@jax.jit
def kernel(x, pos, batch, x_skip, pos_skip, batch_skip, W, b):
    t, idx, wn = _tc_stage(x, pos, batch, x_skip, pos_skip, batch_skip, W, b)

    idx_flat = idx.reshape(M_F * 4)
    wrep = jnp.broadcast_to(wn.reshape(M_F * 4, 1),
                            (M_F * 4, SC_LANES))

    mesh = plsc.VectorSubcoreMesh(core_axis_name="c", subcore_axis_name="s")
    sc = pl.kernel(
        _sc_body,
        out_type=jax.ShapeDtypeStruct((M_F, D_OUT), jnp.float32),
        mesh=mesh,
        scratch_types=[
            pltpu.VMEM((SC_R,), jnp.int32),
            pltpu.VMEM((SC_R,), jnp.int32),
            pltpu.VMEM((SC_R, D_OUT), jnp.float32),
            pltpu.VMEM((SC_R, D_OUT), jnp.float32),
            pltpu.VMEM((SC_R, SC_LANES), jnp.float32),
            pltpu.VMEM((SC_R, SC_LANES), jnp.float32),
            pltpu.VMEM((SC_CP, D_OUT), jnp.float32),
            pltpu.VMEM((SC_CP, D_OUT), jnp.float32),
            pltpu.SemaphoreType.DMA,
            pltpu.SemaphoreType.DMA,
            pltpu.SemaphoreType.DMA,
            pltpu.SemaphoreType.DMA,
            pltpu.SemaphoreType.DMA,
            pltpu.SemaphoreType.DMA,
            pltpu.SemaphoreType.DMA,
            pltpu.SemaphoreType.DMA,
        ],
    )
    y = sc(t, idx_flat, wrep)
    return y


# speedup vs baseline: 1.4861x; 1.4861x over previous
import jax
import jax.numpy as jnp
from jax import lax
from jax.experimental import pallas as pl
from jax.experimental.pallas import tpu as pltpu
from jax.experimental.pallas import tpu_sc as plsc

N_C = 4096
M_F = 16384
M_H = M_F // 2
D_IN = 512
D_SKIP = 256
D_OUT = 512
BM = 256
N_XW = N_C // BM
N_FB = M_H // BM

SC_LANES = 16
SC_NW = 32
SC_P = M_H // SC_NW
SC_CP = 16
SC_R = 3 * SC_CP


def _knn(posT_ref, batchf_ref, ps_ref, bsf_ref, xs_ref, w2t_ref, b_ref,
         idx_ref, wn_ref, skip_ref):
    q0 = ps_ref[:, 0:1]
    q1 = ps_ref[:, 1:2]
    q2 = ps_ref[:, 2:3]
    p0 = posT_ref[0:1, :]
    p1 = posT_ref[1:2, :]
    p2 = posT_ref[2:3, :]
    d2 = (q0 - p0) ** 2 + (q1 - p1) ** 2 + (q2 - p2) ** 2
    same = bsf_ref[...] == batchf_ref[...]
    masked = jnp.where(same, d2, jnp.inf)

    iota = jax.lax.broadcasted_iota(jnp.int32, (1, N_C), 1).astype(jnp.float32)
    idx_cols = []
    w_cols = []
    wsum = jnp.zeros((BM, 1), jnp.float32)
    for k in range(3):
        mval = jnp.min(masked, axis=1, keepdims=True)
        tie = jnp.where(masked == mval, iota, float(N_C))
        midx = jnp.min(tie, axis=1, keepdims=True)
        w = 1.0 / (jnp.sqrt(mval) + 1e-8)
        idx_cols.append(midx)
        w_cols.append(w)
        wsum = wsum + w
        if k < 2:
            hit = iota == midx
            masked = jnp.where(hit, jnp.inf, masked)

    inv = 1.0 / (wsum + 1e-8)
    idx_ref[...] = jnp.concatenate(idx_cols, axis=1).astype(jnp.int32)
    wn_ref[...] = jnp.concatenate(
        [w_cols[0] * inv, w_cols[1] * inv, w_cols[2] * inv], axis=1)
    skip_ref[...] = (jnp.dot(xs_ref[...].astype(jnp.bfloat16), w2t_ref[...],
                             preferred_element_type=jnp.float32)
                     + b_ref[...])


def _tc1_body(posT_ref, batchf_ref, x_ref, ps_ref, bsf_ref, xs_ref,
              w1t_ref, w2t_ref, b_ref, xw_ref, idx_ref, wn_ref, skip_ref):
    i = pl.program_id(0)

    @pl.when(i < N_XW)
    def _xw():
        xw_ref[...] = jnp.dot(x_ref[...].astype(jnp.bfloat16), w1t_ref[...],
                              preferred_element_type=jnp.float32)

    @pl.when(i >= N_XW)
    def _k():
        _knn(posT_ref, batchf_ref, ps_ref, bsf_ref, xs_ref, w2t_ref, b_ref,
             idx_ref, wn_ref, skip_ref)


def _tc2_body(posT_ref, batchf_ref, ps_ref, bsf_ref, xs_ref,
              w2t_ref, b_ref, idx_ref, wn_ref, skip_ref):
    _knn(posT_ref, batchf_ref, ps_ref, bsf_ref, xs_ref, w2t_ref, b_ref,
         idx_ref, wn_ref, skip_ref)


def _sc_body(xw_hbm, idx_hbm, wrep_hbm, skip_hbm, out_hbm,
             idx0, idx1, rows0, rows1, wrep0, wrep1, skp0, skp1, acc0, acc1,
             semg0, semg1, semo0, semo1, semi0, semi1, semw0, semw1,
             sems0, sems1):
    wid = lax.axis_index("s") * 2 + lax.axis_index("c")
    nchunk = SC_P // SC_CP
    half = nchunk // 2

    def aux_slice(g):
        return pl.ds(3 * (wid * SC_P + g * SC_CP), SC_R)

    def pt_slice(g):
        return pl.ds(wid * SC_P + g * SC_CP, SC_CP)

    def compute(rows_v, wrep_v, skp_v, acc_v):
        def point(p, c):
            w0 = wrep_v[3 * p, :]
            w1 = wrep_v[3 * p + 1, :]
            w2 = wrep_v[3 * p + 2, :]
            for v in range(D_OUT // SC_LANES):
                sl = pl.ds(SC_LANES * v, SC_LANES)
                acc_v[p, sl] = (skp_v[p, sl]
                                + w0 * rows_v[3 * p, sl]
                                + w1 * rows_v[3 * p + 1, sl]
                                + w2 * rows_v[3 * p + 2, sl])
            return c

        lax.fori_loop(0, SC_CP, point, 0)

    pltpu.sync_copy(idx_hbm.at[aux_slice(0)], idx0)
    pltpu.sync_copy(wrep_hbm.at[aux_slice(0)], wrep0)
    pltpu.sync_copy(skip_hbm.at[pt_slice(0)], skp0)
    pltpu.async_copy(xw_hbm.at[idx0], rows0, semg0)
    pltpu.sync_copy(idx_hbm.at[aux_slice(1)], idx1)
    pltpu.sync_copy(wrep_hbm.at[aux_slice(1)], wrep1)
    pltpu.sync_copy(skip_hbm.at[pt_slice(1)], skp1)
    pltpu.async_copy(xw_hbm.at[idx1], rows1, semg1)

    def phase(h, g, idx_v, rows_v, wrep_v, skp_v, acc_v,
              semg, semo, semi, semw, sems):
        @pl.when(h > 0)
        def _():
            pltpu.make_async_copy(acc_v, out_hbm.at[pt_slice(g)], semo).wait()

        pltpu.make_async_copy(xw_hbm.at[idx_v], rows_v, semg).wait()

        @pl.when(h < half - 1)
        def _():
            pltpu.async_copy(idx_hbm.at[aux_slice(g + 2)], idx_v, semi)

        @pl.when(h > 0)
        def _():
            pltpu.make_async_copy(wrep_hbm.at[aux_slice(g)], wrep_v,
                                  semw).wait()
            pltpu.make_async_copy(skip_hbm.at[pt_slice(g)], skp_v,
                                  sems).wait()

        compute(rows_v, wrep_v, skp_v, acc_v)
        pltpu.async_copy(acc_v, out_hbm.at[pt_slice(g)], semo)

        @pl.when(h < half - 1)
        def _():
            pltpu.make_async_copy(idx_hbm.at[aux_slice(g + 2)], idx_v,
                                  semi).wait()
            pltpu.async_copy(xw_hbm.at[idx_v], rows_v, semg)
            pltpu.async_copy(wrep_hbm.at[aux_slice(g + 2)], wrep_v, semw)
            pltpu.async_copy(skip_hbm.at[pt_slice(g + 2)], skp_v, sems)

    def body(h, carry):
        phase(h, 2 * h, idx0, rows0, wrep0, skp0, acc0,
              semg0, semo0, semi0, semw0, sems0)
        phase(h, 2 * h + 1, idx1, rows1, wrep1, skp1, acc1,
              semg1, semo1, semi1, semw1, sems1)
        return carry

    lax.fori_loop(0, half, body, 0)
    pltpu.make_async_copy(acc0, out_hbm.at[pt_slice(nchunk - 2)], semo0).wait()
    pltpu.make_async_copy(acc1, out_hbm.at[pt_slice(nchunk - 1)], semo1).wait()


def _make_sc():
    mesh = plsc.VectorSubcoreMesh(core_axis_name="c", subcore_axis_name="s")
    return pl.kernel(
        _sc_body,
        out_type=jax.ShapeDtypeStruct((M_H, D_OUT), jnp.float32),
        mesh=mesh,
        scratch_types=[
            pltpu.VMEM((SC_R,), jnp.int32),
            pltpu.VMEM((SC_R,), jnp.int32),
            pltpu.VMEM((SC_R, D_OUT), jnp.float32),
            pltpu.VMEM((SC_R, D_OUT), jnp.float32),
            pltpu.VMEM((SC_R, SC_LANES), jnp.float32),
            pltpu.VMEM((SC_R, SC_LANES), jnp.float32),
            pltpu.VMEM((SC_CP, D_OUT), jnp.float32),
            pltpu.VMEM((SC_CP, D_OUT), jnp.float32),
            pltpu.VMEM((SC_CP, D_OUT), jnp.float32),
            pltpu.VMEM((SC_CP, D_OUT), jnp.float32),
        ] + [pltpu.SemaphoreType.DMA] * 10,
    )


@jax.jit
def kernel(x, pos, batch, x_skip, pos_skip, batch_skip, W, b):
    posT = pos.T
    batchf = batch.astype(jnp.float32).reshape(1, N_C)
    bsf = batch_skip.astype(jnp.float32).reshape(M_F, 1)
    WT = W.T.astype(jnp.bfloat16)
    w1t = WT[:D_IN]
    w2t = WT[D_IN:]
    b2 = b.reshape(1, D_OUT)

    fine1 = lambda i: (jnp.maximum(i - N_XW, 0), 0)
    cst = lambda i: (0, 0)
    xw, idx1, wn1, skip1 = pl.pallas_call(
        _tc1_body,
        grid=(N_XW + N_FB,),
        in_specs=[
            pl.BlockSpec((3, N_C), cst),
            pl.BlockSpec((1, N_C), cst),
            pl.BlockSpec((BM, D_IN), lambda i: (jnp.minimum(i, N_XW - 1), 0)),
            pl.BlockSpec((BM, 3), fine1),
            pl.BlockSpec((BM, 1), fine1),
            pl.BlockSpec((BM, D_SKIP), fine1),
            pl.BlockSpec((D_IN, D_OUT), cst),
            pl.BlockSpec((D_SKIP, D_OUT), cst),
            pl.BlockSpec((1, D_OUT), cst),
        ],
        out_specs=[
            pl.BlockSpec((BM, D_OUT), lambda i: (jnp.minimum(i, N_XW - 1), 0)),
            pl.BlockSpec((BM, 3), fine1),
            pl.BlockSpec((BM, 3), fine1),
            pl.BlockSpec((BM, D_OUT), fine1),
        ],
        out_shape=[
            jax.ShapeDtypeStruct((N_C, D_IN), jnp.float32),
            jax.ShapeDtypeStruct((M_H, 3), jnp.int32),
            jax.ShapeDtypeStruct((M_H, 3), jnp.float32),
            jax.ShapeDtypeStruct((M_H, D_OUT), jnp.float32),
        ],
    )(posT, batchf, x, pos_skip[:M_H], bsf[:M_H], x_skip[:M_H],
      w1t, w2t, b2)

    blk = lambda i: (i, 0)
    idx2, wn2, skip2 = pl.pallas_call(
        _tc2_body,
        grid=(N_FB,),
        in_specs=[
            pl.BlockSpec((3, N_C), cst),
            pl.BlockSpec((1, N_C), cst),
            pl.BlockSpec((BM, 3), blk),
            pl.BlockSpec((BM, 1), blk),
            pl.BlockSpec((BM, D_SKIP), blk),
            pl.BlockSpec((D_SKIP, D_OUT), cst),
            pl.BlockSpec((1, D_OUT), cst),
        ],
        out_specs=[
            pl.BlockSpec((BM, 3), blk),
            pl.BlockSpec((BM, 3), blk),
            pl.BlockSpec((BM, D_OUT), blk),
        ],
        out_shape=[
            jax.ShapeDtypeStruct((M_H, 3), jnp.int32),
            jax.ShapeDtypeStruct((M_H, 3), jnp.float32),
            jax.ShapeDtypeStruct((M_H, D_OUT), jnp.float32),
        ],
    )(posT, batchf, pos_skip[M_H:], bsf[M_H:], x_skip[M_H:], w2t, b2)

    sc = _make_sc()
    y_halves = []
    for idx_h, wn_h, skip_h in ((idx1, wn1, skip1), (idx2, wn2, skip2)):
        idx_flat = idx_h.reshape(M_H * 3)
        wrep = jnp.broadcast_to(wn_h.reshape(M_H * 3, 1),
                                (M_H * 3, SC_LANES))
        y_halves.append(sc(xw, idx_flat, wrep, skip_h))
    return jnp.concatenate(y_halves, axis=0)


# speedup vs baseline: 1.4861x; 1.0000x over previous
import jax
import jax.numpy as jnp
from jax import lax
from jax.experimental import pallas as pl
from jax.experimental.pallas import tpu as pltpu
from jax.experimental.pallas import tpu_sc as plsc

N_C = 4096
M_F = 16384
M_H = M_F // 2
D_IN = 512
D_SKIP = 256
D_OUT = 512
BM = 512
N_XW = N_C // BM
N_FB = M_H // BM
NCB = 2048
N_J = N_C // NCB

SC_LANES = 16
SC_NW = 32
SC_P = M_H // SC_NW
SC_CP = 16
SC_R = 3 * SC_CP


def _insert(x, xi, v1, v2, v3, i1, i2, i3):
    c1 = x < v1
    c2 = x < v2
    c3 = x < v3
    nv1 = jnp.where(c1, x, v1)
    ni1 = jnp.where(c1, xi, i1)
    nv2 = jnp.where(c1, v1, jnp.where(c2, x, v2))
    ni2 = jnp.where(c1, i1, jnp.where(c2, xi, i2))
    nv3 = jnp.where(c2, v2, jnp.where(c3, x, v3))
    ni3 = jnp.where(c2, i2, jnp.where(c3, xi, i3))
    return nv1, nv2, nv3, ni1, ni2, ni3


def _knn_step(j, lo_i, hi_i, posT_ref, batchf_ref, ps_ref, bsf_ref, xs_ref,
              w2t_ref, b_ref, idx_ref, wn_ref, skip_ref,
              v1_s, v2_s, v3_s, i1_s, i2_s, i3_s):
    @pl.when(j == lo_i)
    def _init():
        inf2 = jnp.full((BM, 1), jnp.inf, jnp.float32)
        v1_s[...] = inf2
        v2_s[...] = inf2
        v3_s[...] = inf2
        i1_s[...] = inf2
        i2_s[...] = inf2
        i3_s[...] = inf2
        skip_ref[...] = (jnp.dot(xs_ref[...].astype(jnp.bfloat16),
                                 w2t_ref[...],
                                 preferred_element_type=jnp.float32)
                         + b_ref[...])

    q0 = ps_ref[:, 0:1]
    q1 = ps_ref[:, 1:2]
    q2 = ps_ref[:, 2:3]
    p0 = posT_ref[0:1, :]
    p1 = posT_ref[1:2, :]
    p2 = posT_ref[2:3, :]
    d2 = (q0 - p0) ** 2 + (q1 - p1) ** 2 + (q2 - p2) ** 2
    same = bsf_ref[...] == batchf_ref[...]
    masked = jnp.where(same, d2, jnp.inf)

    iota = (jax.lax.broadcasted_iota(jnp.int32, (1, NCB), 1)
            .astype(jnp.float32) + (j * NCB).astype(jnp.float32))
    v1 = v1_s[...]
    v2 = v2_s[...]
    v3 = v3_s[...]
    i1 = i1_s[...]
    i2 = i2_s[...]
    i3 = i3_s[...]
    for k in range(3):
        mval = jnp.min(masked, axis=1, keepdims=True)
        tie = jnp.where(masked == mval, iota, jnp.inf)
        midx = jnp.min(tie, axis=1, keepdims=True)
        v1, v2, v3, i1, i2, i3 = _insert(mval, midx, v1, v2, v3, i1, i2, i3)
        if k < 2:
            hit = iota == midx
            masked = jnp.where(hit, jnp.inf, masked)
    v1_s[...] = v1
    v2_s[...] = v2
    v3_s[...] = v3
    i1_s[...] = i1
    i2_s[...] = i2
    i3_s[...] = i3

    @pl.when(j == hi_i - 1)
    def _finalize():
        w1 = 1.0 / (jnp.sqrt(v1) + 1e-8)
        w2 = 1.0 / (jnp.sqrt(v2) + 1e-8)
        w3 = 1.0 / (jnp.sqrt(v3) + 1e-8)
        inv = 1.0 / (w1 + w2 + w3 + 1e-8)
        safe = lambda ix: jnp.where(jnp.isfinite(ix), ix, 0.0)
        idx_ref[...] = jnp.concatenate(
            [safe(i1), safe(i2), safe(i3)], axis=1).astype(jnp.int32)
        wn_ref[...] = jnp.concatenate(
            [w1 * inv, w2 * inv, w3 * inv], axis=1)


def _tc1_body(lo_ref, hi_ref, posT_ref, batchf_ref, x_ref, ps_ref, bsf_ref,
              xs_ref, w1t_ref, w2t_ref, b_ref, xw_ref, idx_ref, wn_ref,
              skip_ref, v1_s, v2_s, v3_s, i1_s, i2_s, i3_s):
    i = pl.program_id(0)
    j = pl.program_id(1)
    lo_i = lo_ref[i]
    hi_i = hi_ref[i]

    @pl.when(jnp.logical_and(i < N_XW, j == 0))
    def _xw():
        xw_ref[...] = jnp.dot(x_ref[...].astype(jnp.bfloat16), w1t_ref[...],
                              preferred_element_type=jnp.float32)

    @pl.when(jnp.logical_and(i >= N_XW,
                             jnp.logical_and(j >= lo_i, j < hi_i)))
    def _k():
        _knn_step(j, lo_i, hi_i, posT_ref, batchf_ref, ps_ref, bsf_ref,
                  xs_ref, w2t_ref, b_ref, idx_ref, wn_ref, skip_ref,
                  v1_s, v2_s, v3_s, i1_s, i2_s, i3_s)


def _tc2_body(lo_ref, hi_ref, posT_ref, batchf_ref, ps_ref, bsf_ref, xs_ref,
              w2t_ref, b_ref, idx_ref, wn_ref, skip_ref,
              v1_s, v2_s, v3_s, i1_s, i2_s, i3_s):
    i = pl.program_id(0)
    j = pl.program_id(1)
    lo_i = lo_ref[i]
    hi_i = hi_ref[i]

    @pl.when(jnp.logical_and(j >= lo_i, j < hi_i))
    def _k():
        _knn_step(j, lo_i, hi_i, posT_ref, batchf_ref, ps_ref, bsf_ref,
                  xs_ref, w2t_ref, b_ref, idx_ref, wn_ref, skip_ref,
                  v1_s, v2_s, v3_s, i1_s, i2_s, i3_s)


def _sc_body(xw_hbm, idx_hbm, wrep_hbm, skip_hbm, out_hbm,
             idx0, idx1, rows0, rows1, wrep0, wrep1, skp0, skp1, acc0, acc1,
             semg0, semg1, semo0, semo1, semi0, semi1, semw0, semw1,
             sems0, sems1):
    wid = lax.axis_index("s") * 2 + lax.axis_index("c")
    nchunk = SC_P // SC_CP
    half = nchunk // 2

    def aux_slice(g):
        return pl.ds(3 * (wid * SC_P + g * SC_CP), SC_R)

    def pt_slice(g):
        return pl.ds(wid * SC_P + g * SC_CP, SC_CP)

    def compute(rows_v, wrep_v, skp_v, acc_v):
        def point(p, c):
            w0 = wrep_v[3 * p, :]
            w1 = wrep_v[3 * p + 1, :]
            w2 = wrep_v[3 * p + 2, :]
            for v in range(D_OUT // SC_LANES):
                sl = pl.ds(SC_LANES * v, SC_LANES)
                acc_v[p, sl] = (skp_v[p, sl]
                                + w0 * rows_v[3 * p, sl]
                                + w1 * rows_v[3 * p + 1, sl]
                                + w2 * rows_v[3 * p + 2, sl])
            return c

        lax.fori_loop(0, SC_CP, point, 0)

    pltpu.sync_copy(idx_hbm.at[aux_slice(0)], idx0)
    pltpu.sync_copy(wrep_hbm.at[aux_slice(0)], wrep0)
    pltpu.sync_copy(skip_hbm.at[pt_slice(0)], skp0)
    pltpu.async_copy(xw_hbm.at[idx0], rows0, semg0)
    pltpu.sync_copy(idx_hbm.at[aux_slice(1)], idx1)
    pltpu.sync_copy(wrep_hbm.at[aux_slice(1)], wrep1)
    pltpu.sync_copy(skip_hbm.at[pt_slice(1)], skp1)
    pltpu.async_copy(xw_hbm.at[idx1], rows1, semg1)

    def phase(h, g, idx_v, rows_v, wrep_v, skp_v, acc_v,
              semg, semo, semi, semw, sems):
        @pl.when(h > 0)
        def _():
            pltpu.make_async_copy(acc_v, out_hbm.at[pt_slice(g)], semo).wait()

        pltpu.make_async_copy(xw_hbm.at[idx_v], rows_v, semg).wait()

        @pl.when(h < half - 1)
        def _():
            pltpu.async_copy(idx_hbm.at[aux_slice(g + 2)], idx_v, semi)

        @pl.when(h > 0)
        def _():
            pltpu.make_async_copy(wrep_hbm.at[aux_slice(g)], wrep_v,
                                  semw).wait()
            pltpu.make_async_copy(skip_hbm.at[pt_slice(g)], skp_v,
                                  sems).wait()

        compute(rows_v, wrep_v, skp_v, acc_v)
        pltpu.async_copy(acc_v, out_hbm.at[pt_slice(g)], semo)

        @pl.when(h < half - 1)
        def _():
            pltpu.make_async_copy(idx_hbm.at[aux_slice(g + 2)], idx_v,
                                  semi).wait()
            pltpu.async_copy(xw_hbm.at[idx_v], rows_v, semg)
            pltpu.async_copy(wrep_hbm.at[aux_slice(g + 2)], wrep_v, semw)
            pltpu.async_copy(skip_hbm.at[pt_slice(g + 2)], skp_v, sems)

    def body(h, carry):
        phase(h, 2 * h, idx0, rows0, wrep0, skp0, acc0,
              semg0, semo0, semi0, semw0, sems0)
        phase(h, 2 * h + 1, idx1, rows1, wrep1, skp1, acc1,
              semg1, semo1, semi1, semw1, sems1)
        return carry

    lax.fori_loop(0, half, body, 0)
    pltpu.make_async_copy(acc0, out_hbm.at[pt_slice(nchunk - 2)], semo0).wait()
    pltpu.make_async_copy(acc1, out_hbm.at[pt_slice(nchunk - 1)], semo1).wait()


def _make_sc():
    mesh = plsc.VectorSubcoreMesh(core_axis_name="c", subcore_axis_name="s",
                                  num_cores=2, num_subcores=16)
    return pl.kernel(
        _sc_body,
        out_type=jax.ShapeDtypeStruct((M_H, D_OUT), jnp.float32),
        mesh=mesh,
        scratch_types=[
            pltpu.VMEM((SC_R,), jnp.int32),
            pltpu.VMEM((SC_R,), jnp.int32),
            pltpu.VMEM((SC_R, D_OUT), jnp.float32),
            pltpu.VMEM((SC_R, D_OUT), jnp.float32),
            pltpu.VMEM((SC_R, SC_LANES), jnp.float32),
            pltpu.VMEM((SC_R, SC_LANES), jnp.float32),
            pltpu.VMEM((SC_CP, D_OUT), jnp.float32),
            pltpu.VMEM((SC_CP, D_OUT), jnp.float32),
            pltpu.VMEM((SC_CP, D_OUT), jnp.float32),
            pltpu.VMEM((SC_CP, D_OUT), jnp.float32),
        ] + [pltpu.SemaphoreType.DMA] * 10,
    )


@jax.jit
def kernel(x, pos, batch, x_skip, pos_skip, batch_skip, W, b):
    posT = pos.T
    batchf = batch.astype(jnp.float32).reshape(1, N_C)
    bsf = batch_skip.astype(jnp.float32).reshape(M_F, 1)
    WT = W.T.astype(jnp.bfloat16)
    w1t = WT[:D_IN]
    w2t = WT[D_IN:]
    b2 = b.reshape(1, D_OUT)

    bvals = jnp.arange(8, dtype=batch.dtype)
    cs = jnp.searchsorted(batch, bvals, side="left").astype(jnp.int32)
    ce = jnp.searchsorted(batch, bvals, side="right").astype(jnp.int32)

    def windows(bs_half):
        bs2 = bs_half.reshape(N_FB, BM)
        lo_f = (cs[bs2[:, 0]] // NCB).astype(jnp.int32)
        hi_f = ((ce[bs2[:, -1]] + NCB - 1) // NCB).astype(jnp.int32)
        lo_f = jnp.minimum(lo_f, N_J - 1)
        hi_f = jnp.minimum(jnp.maximum(hi_f, lo_f + 1), N_J)
        return lo_f, hi_f

    lo1_f, hi1_f = windows(batch_skip[:M_H])
    lo2, hi2 = windows(batch_skip[M_H:])
    lo1 = jnp.concatenate([jnp.zeros((N_XW,), jnp.int32), lo1_f])
    hi1 = jnp.concatenate([jnp.ones((N_XW,), jnp.int32), hi1_f])

    fine1 = lambda i, j, lo_r, hi_r: (jnp.maximum(i - N_XW, 0), 0)
    cst = lambda i, j, lo_r, hi_r: (0, 0)
    swp = lambda i, j, lo_r, hi_r: (0, j)
    scratch6 = [pltpu.VMEM((BM, 1), jnp.float32)] * 6
    xw, idx1, wn1, skip1 = pl.pallas_call(
        _tc1_body,
        grid_spec=pltpu.PrefetchScalarGridSpec(
            num_scalar_prefetch=2,
            grid=(N_XW + N_FB, N_J),
            in_specs=[
                pl.BlockSpec((3, NCB), swp),
                pl.BlockSpec((1, NCB), swp),
                pl.BlockSpec((BM, D_IN),
                             lambda i, j, lo_r, hi_r:
                             (jnp.minimum(i, N_XW - 1), 0)),
                pl.BlockSpec((BM, 3), fine1),
                pl.BlockSpec((BM, 1), fine1),
                pl.BlockSpec((BM, D_SKIP), fine1),
                pl.BlockSpec((D_IN, D_OUT), cst),
                pl.BlockSpec((D_SKIP, D_OUT), cst),
                pl.BlockSpec((1, D_OUT), cst),
            ],
            out_specs=[
                pl.BlockSpec((BM, D_OUT),
                             lambda i, j, lo_r, hi_r:
                             (jnp.minimum(i, N_XW - 1), 0)),
                pl.BlockSpec((BM, 3), fine1),
                pl.BlockSpec((BM, 3), fine1),
                pl.BlockSpec((BM, D_OUT), fine1),
            ],
            scratch_shapes=scratch6,
        ),
        out_shape=[
            jax.ShapeDtypeStruct((N_C, D_IN), jnp.float32),
            jax.ShapeDtypeStruct((M_H, 3), jnp.int32),
            jax.ShapeDtypeStruct((M_H, 3), jnp.float32),
            jax.ShapeDtypeStruct((M_H, D_OUT), jnp.float32),
        ],
    )(lo1, hi1, posT, batchf, x, pos_skip[:M_H], bsf[:M_H], x_skip[:M_H],
      w1t, w2t, b2)

    blk = lambda i, j, lo_r, hi_r: (i, 0)
    idx2, wn2, skip2 = pl.pallas_call(
        _tc2_body,
        grid_spec=pltpu.PrefetchScalarGridSpec(
            num_scalar_prefetch=2,
            grid=(N_FB, N_J),
            in_specs=[
                pl.BlockSpec((3, NCB), swp),
                pl.BlockSpec((1, NCB), swp),
                pl.BlockSpec((BM, 3), blk),
                pl.BlockSpec((BM, 1), blk),
                pl.BlockSpec((BM, D_SKIP), blk),
                pl.BlockSpec((D_SKIP, D_OUT), cst),
                pl.BlockSpec((1, D_OUT), cst),
            ],
            out_specs=[
                pl.BlockSpec((BM, 3), blk),
                pl.BlockSpec((BM, 3), blk),
                pl.BlockSpec((BM, D_OUT), blk),
            ],
            scratch_shapes=scratch6,
        ),
        out_shape=[
            jax.ShapeDtypeStruct((M_H, 3), jnp.int32),
            jax.ShapeDtypeStruct((M_H, 3), jnp.float32),
            jax.ShapeDtypeStruct((M_H, D_OUT), jnp.float32),
        ],
    )(lo2, hi2, posT, batchf, pos_skip[M_H:], bsf[M_H:], x_skip[M_H:],
      w2t, b2)

    sc = _make_sc()
    y_halves = []
    for idx_h, wn_h, skip_h in ((idx1, wn1, skip1), (idx2, wn2, skip2)):
        idx_flat = idx_h.reshape(M_H * 3)
        wrep = jnp.broadcast_to(wn_h.reshape(M_H * 3, 1),
                                (M_H * 3, SC_LANES))
        y_halves.append(sc(xw, idx_flat, wrep, skip_h))
    return jnp.concatenate(y_halves, axis=0)


# speedup vs baseline: 1.4890x; 1.0019x over previous
import jax
import jax.numpy as jnp
from jax import lax
from jax.experimental import pallas as pl
from jax.experimental.pallas import tpu as pltpu
from jax.experimental.pallas import tpu_sc as plsc

N_C = 4096
M_F = 16384
M_H = M_F // 2
D_IN = 512
D_SKIP = 256
D_OUT = 512
BM = 256
N_XW = N_C // BM
N_FB = M_H // BM

SC_LANES = 16
SC_NW = 32
SC_P = M_H // SC_NW
SC_CP = 16
SC_R = 3 * SC_CP


def _knn(posT_ref, batchf_ref, ps_ref, bsf_ref, xs_ref, w2t_ref, b_ref,
         idx_ref, wn_ref, skip_ref):
    q0 = ps_ref[:, 0:1]
    q1 = ps_ref[:, 1:2]
    q2 = ps_ref[:, 2:3]
    p0 = posT_ref[0:1, :]
    p1 = posT_ref[1:2, :]
    p2 = posT_ref[2:3, :]
    d2 = (q0 - p0) ** 2 + (q1 - p1) ** 2 + (q2 - p2) ** 2
    same = bsf_ref[...] == batchf_ref[...]
    masked = jnp.where(same, d2, jnp.inf)

    iota = jax.lax.broadcasted_iota(jnp.int32, (1, N_C), 1).astype(jnp.float32)
    idx_cols = []
    w_cols = []
    wsum = jnp.zeros((BM, 1), jnp.float32)
    for k in range(3):
        mval = jnp.min(masked, axis=1, keepdims=True)
        tie = jnp.where(masked == mval, iota, float(N_C))
        midx = jnp.min(tie, axis=1, keepdims=True)
        w = 1.0 / (jnp.sqrt(mval) + 1e-8)
        idx_cols.append(midx)
        w_cols.append(w)
        wsum = wsum + w
        if k < 2:
            hit = iota == midx
            masked = jnp.where(hit, jnp.inf, masked)

    inv = 1.0 / (wsum + 1e-8)
    idx_ref[...] = jnp.concatenate(idx_cols, axis=1).astype(jnp.int32)
    wn_ref[...] = jnp.concatenate(
        [w_cols[0] * inv, w_cols[1] * inv, w_cols[2] * inv], axis=1)
    skip_ref[...] = (jnp.dot(xs_ref[...].astype(jnp.bfloat16), w2t_ref[...],
                             preferred_element_type=jnp.float32)
                     + b_ref[...])


def _tc1_body(posT_ref, batchf_ref, x_ref, ps_ref, bsf_ref, xs_ref,
              w1t_ref, w2t_ref, b_ref, xw_ref, idx_ref, wn_ref, skip_ref):
    i = pl.program_id(0)

    @pl.when(i < N_XW)
    def _xw():
        xw_ref[...] = jnp.dot(x_ref[...].astype(jnp.bfloat16), w1t_ref[...],
                              preferred_element_type=jnp.float32)

    @pl.when(i >= N_XW)
    def _k():
        _knn(posT_ref, batchf_ref, ps_ref, bsf_ref, xs_ref, w2t_ref, b_ref,
             idx_ref, wn_ref, skip_ref)


def _tc2_body(posT_ref, batchf_ref, ps_ref, bsf_ref, xs_ref,
              w2t_ref, b_ref, idx_ref, wn_ref, skip_ref):
    _knn(posT_ref, batchf_ref, ps_ref, bsf_ref, xs_ref, w2t_ref, b_ref,
         idx_ref, wn_ref, skip_ref)


def _sc_body(xw_hbm, idx_hbm, wrep_hbm, skip_hbm, out_hbm,
             idx0, idx1, rows0, rows1, wrep0, wrep1, skp0, skp1, acc0, acc1,
             semg0, semg1, semo0, semo1, semi0, semi1, semw0, semw1,
             sems0, sems1):
    wid = lax.axis_index("s") * 2 + lax.axis_index("c")
    nchunk = SC_P // SC_CP
    half = nchunk // 2

    def aux_slice(g):
        return pl.ds(3 * (wid * SC_P + g * SC_CP), SC_R)

    def pt_slice(g):
        return pl.ds(wid * SC_P + g * SC_CP, SC_CP)

    def compute(rows_v, wrep_v, skp_v, acc_v):
        def point(p, c):
            w0 = wrep_v[3 * p, :]
            w1 = wrep_v[3 * p + 1, :]
            w2 = wrep_v[3 * p + 2, :]
            for v in range(D_OUT // SC_LANES):
                sl = pl.ds(SC_LANES * v, SC_LANES)
                acc_v[p, sl] = (skp_v[p, sl]
                                + w0 * rows_v[3 * p, sl]
                                + w1 * rows_v[3 * p + 1, sl]
                                + w2 * rows_v[3 * p + 2, sl])
            return c

        lax.fori_loop(0, SC_CP, point, 0)

    pltpu.sync_copy(idx_hbm.at[aux_slice(0)], idx0)
    pltpu.sync_copy(wrep_hbm.at[aux_slice(0)], wrep0)
    pltpu.sync_copy(skip_hbm.at[pt_slice(0)], skp0)
    pltpu.async_copy(xw_hbm.at[idx0], rows0, semg0)
    pltpu.sync_copy(idx_hbm.at[aux_slice(1)], idx1)
    pltpu.sync_copy(wrep_hbm.at[aux_slice(1)], wrep1)
    pltpu.sync_copy(skip_hbm.at[pt_slice(1)], skp1)
    pltpu.async_copy(xw_hbm.at[idx1], rows1, semg1)

    def phase(h, g, idx_v, rows_v, wrep_v, skp_v, acc_v,
              semg, semo, semi, semw, sems):
        @pl.when(h > 0)
        def _():
            pltpu.make_async_copy(acc_v, out_hbm.at[pt_slice(g)], semo).wait()

        pltpu.make_async_copy(xw_hbm.at[idx_v], rows_v, semg).wait()

        @pl.when(h < half - 1)
        def _():
            pltpu.async_copy(idx_hbm.at[aux_slice(g + 2)], idx_v, semi)

        @pl.when(h > 0)
        def _():
            pltpu.make_async_copy(wrep_hbm.at[aux_slice(g)], wrep_v,
                                  semw).wait()
            pltpu.make_async_copy(skip_hbm.at[pt_slice(g)], skp_v,
                                  sems).wait()

        compute(rows_v, wrep_v, skp_v, acc_v)
        pltpu.async_copy(acc_v, out_hbm.at[pt_slice(g)], semo)

        @pl.when(h < half - 1)
        def _():
            pltpu.make_async_copy(idx_hbm.at[aux_slice(g + 2)], idx_v,
                                  semi).wait()
            pltpu.async_copy(xw_hbm.at[idx_v], rows_v, semg)
            pltpu.async_copy(wrep_hbm.at[aux_slice(g + 2)], wrep_v, semw)
            pltpu.async_copy(skip_hbm.at[pt_slice(g + 2)], skp_v, sems)

    def body(h, carry):
        phase(h, 2 * h, idx0, rows0, wrep0, skp0, acc0,
              semg0, semo0, semi0, semw0, sems0)
        phase(h, 2 * h + 1, idx1, rows1, wrep1, skp1, acc1,
              semg1, semo1, semi1, semw1, sems1)
        return carry

    lax.fori_loop(0, half, body, 0)
    pltpu.make_async_copy(acc0, out_hbm.at[pt_slice(nchunk - 2)], semo0).wait()
    pltpu.make_async_copy(acc1, out_hbm.at[pt_slice(nchunk - 1)], semo1).wait()


def _make_sc():
    mesh = plsc.VectorSubcoreMesh(core_axis_name="c", subcore_axis_name="s")
    return pl.kernel(
        _sc_body,
        out_type=jax.ShapeDtypeStruct((M_H, D_OUT), jnp.float32),
        mesh=mesh,
        scratch_types=[
            pltpu.VMEM((SC_R,), jnp.int32),
            pltpu.VMEM((SC_R,), jnp.int32),
            pltpu.VMEM((SC_R, D_OUT), jnp.float32),
            pltpu.VMEM((SC_R, D_OUT), jnp.float32),
            pltpu.VMEM((SC_R, SC_LANES), jnp.float32),
            pltpu.VMEM((SC_R, SC_LANES), jnp.float32),
            pltpu.VMEM((SC_CP, D_OUT), jnp.float32),
            pltpu.VMEM((SC_CP, D_OUT), jnp.float32),
            pltpu.VMEM((SC_CP, D_OUT), jnp.float32),
            pltpu.VMEM((SC_CP, D_OUT), jnp.float32),
        ] + [pltpu.SemaphoreType.DMA] * 10,
    )


@jax.jit
def kernel(x, pos, batch, x_skip, pos_skip, batch_skip, W, b):
    posT = pos.T
    batchf = batch.astype(jnp.float32).reshape(1, N_C)
    bsf = batch_skip.astype(jnp.float32).reshape(M_F, 1)
    WT = W.T.astype(jnp.bfloat16)
    w1t = WT[:D_IN]
    w2t = WT[D_IN:]
    b2 = b.reshape(1, D_OUT)

    fine1 = lambda i: (jnp.maximum(i - N_XW, 0), 0)
    cst = lambda i: (0, 0)
    xw, idx1, wn1, skip1 = pl.pallas_call(
        _tc1_body,
        grid=(N_XW + N_FB,),
        in_specs=[
            pl.BlockSpec((3, N_C), cst),
            pl.BlockSpec((1, N_C), cst),
            pl.BlockSpec((BM, D_IN), lambda i: (jnp.minimum(i, N_XW - 1), 0)),
            pl.BlockSpec((BM, 3), fine1),
            pl.BlockSpec((BM, 1), fine1),
            pl.BlockSpec((BM, D_SKIP), fine1),
            pl.BlockSpec((D_IN, D_OUT), cst),
            pl.BlockSpec((D_SKIP, D_OUT), cst),
            pl.BlockSpec((1, D_OUT), cst),
        ],
        out_specs=[
            pl.BlockSpec((BM, D_OUT), lambda i: (jnp.minimum(i, N_XW - 1), 0)),
            pl.BlockSpec((BM, 3), fine1),
            pl.BlockSpec((BM, 3), fine1),
            pl.BlockSpec((BM, D_OUT), fine1),
        ],
        out_shape=[
            jax.ShapeDtypeStruct((N_C, D_IN), jnp.float32),
            jax.ShapeDtypeStruct((M_H, 3), jnp.int32),
            jax.ShapeDtypeStruct((M_H, 3), jnp.float32),
            jax.ShapeDtypeStruct((M_H, D_OUT), jnp.float32),
        ],
    )(posT, batchf, x, pos_skip[:M_H], bsf[:M_H], x_skip[:M_H],
      w1t, w2t, b2)

    blk = lambda i: (i, 0)
    idx2, wn2, skip2 = pl.pallas_call(
        _tc2_body,
        grid=(N_FB,),
        in_specs=[
            pl.BlockSpec((3, N_C), cst),
            pl.BlockSpec((1, N_C), cst),
            pl.BlockSpec((BM, 3), blk),
            pl.BlockSpec((BM, 1), blk),
            pl.BlockSpec((BM, D_SKIP), blk),
            pl.BlockSpec((D_SKIP, D_OUT), cst),
            pl.BlockSpec((1, D_OUT), cst),
        ],
        out_specs=[
            pl.BlockSpec((BM, 3), blk),
            pl.BlockSpec((BM, 3), blk),
            pl.BlockSpec((BM, D_OUT), blk),
        ],
        out_shape=[
            jax.ShapeDtypeStruct((M_H, 3), jnp.int32),
            jax.ShapeDtypeStruct((M_H, 3), jnp.float32),
            jax.ShapeDtypeStruct((M_H, D_OUT), jnp.float32),
        ],
    )(posT, batchf, pos_skip[M_H:], bsf[M_H:], x_skip[M_H:], w2t, b2)

    sc = _make_sc()
    y_halves = []
    for idx_h, wn_h, skip_h in ((idx1, wn1, skip1), (idx2, wn2, skip2)):
        idx_flat = idx_h.reshape(M_H * 3)
        wrep = jnp.broadcast_to(wn_h.reshape(M_H * 3, 1),
                                (M_H * 3, SC_LANES))
        y_halves.append(sc(xw, idx_flat, wrep, skip_h))
    return jnp.concatenate(y_halves, axis=0)
